# baseline (speedup 1.0000x reference)
import functools

import jax
import jax.numpy as jnp
from jax import lax
from jax.experimental import pallas as pl
from jax.experimental.pallas import tpu as pltpu
from jax.experimental.pallas import tpu_sc as plsc

_B = 1024
_NC = 4
_BC = _B // _NC
_L = 50
_LPAD = 56
_D = 128
_NW = 32
_ROWS = _BC // _NW
_NBUF = 4
_NB = 2048


def _sc_pool_body(w_hbm, ctx_hbm, out_hbm, idx_v, h_v, shared, bufs, sems,
                  hsems):
    sid = lax.axis_index("s")
    wid = sid * 2 + lax.axis_index("c")
    base = wid * _ROWS
    pltpu.sync_copy(ctx_hbm.at[pl.ds(base * _LPAD, _ROWS * _LPAD)], idx_v)

    def issue_row(k, b):
        o = k * _LPAD
        vecs = [idx_v[pl.ds(o, 16)], idx_v[pl.ds(o + 16, 16)],
                idx_v[pl.ds(o + 32, 16)], idx_v[pl.ds(o + 40, 16)]]
        for g in range(_LPAD):
            j, l = (g // 16, g % 16) if g < 48 else (3, g - 40)
            r = vecs[j][l]
            pltpu.async_copy(w_hbm.at[pl.ds(r * _D, _D)],
                             shared.at[sid, b, pl.ds(g * _D, _D)], sems[b])

    def drain(b):
        pltpu.make_async_copy(w_hbm.at[pl.ds(0, _LPAD * _D)],
                              shared.at[sid, b], sems[b]).wait()
        pltpu.async_copy(shared.at[sid, b], bufs[b], hsems[b]).wait()

    for b in range(_NBUF):
        issue_row(b, b)

    def step(i, carry):
        for b in range(_NBUF):
            k = i * _NBUF + b
            drain(b)

            for c in range(_D // 16):
                acc = bufs[b][pl.ds(c * 16, 16)]
                for g in range(1, _LPAD):
                    acc = acc + bufs[b][pl.ds(g * _D + c * 16, 16)]
                h_v[k, pl.ds(c * 16, 16)] = acc

            @pl.when(k + _NBUF < _ROWS)
            def _():
                issue_row(k + _NBUF, b)
        return carry

    lax.fori_loop(0, _ROWS // _NBUF, step, 0)
    pltpu.sync_copy(h_v, out_hbm.at[pl.ds(base, _ROWS)])


_sc_pool = functools.partial(
    pl.kernel,
    out_type=jax.ShapeDtypeStruct((_BC, _D), jnp.float32),
    mesh=plsc.VectorSubcoreMesh(core_axis_name="c", subcore_axis_name="s"),
    scratch_types=[
        pltpu.VMEM((_ROWS * _LPAD,), jnp.int32),
        pltpu.VMEM((_ROWS, _D), jnp.float32),
        pltpu.VMEM_SHARED((16, _NBUF, _LPAD * _D), jnp.float32),
        [pltpu.VMEM((_LPAD * _D,), jnp.float32)] * _NBUF,
        [pltpu.SemaphoreType.DMA] * _NBUF,
        [pltpu.SemaphoreType.DMA] * _NBUF,
    ],
    compiler_params=pltpu.CompilerParams(use_tc_tiling_on_sc=False),
)(_sc_pool_body)


def _tc_proj_body(sums_ref, ctxt_ref, len_ref, w0_ref, wt_ref,
                  out_ref, ht_ref):
    @pl.when(pl.program_id(0) == 0)
    def _():
        n0 = jnp.sum((ctxt_ref[...] == 0).astype(jnp.float32),
                     axis=0)[:, None] + float(_LPAD - _L)
        inv_len = 1.0 / jnp.maximum(len_ref[...], 1).astype(jnp.float32)
        h = (sums_ref[...] - n0 * w0_ref[...]) * inv_len
        ht_ref[...] = jnp.transpose(h).astype(jnp.bfloat16)

    out_ref[...] = jnp.dot(
        wt_ref[...].astype(jnp.bfloat16),
        ht_ref[...],
        preferred_element_type=jnp.float32,
    )


def _tc_proj(prev, chunk, sums, contexts_t, lengths2d, w0, W_out_t):
    v = W_out_t.shape[0]
    grid = (pl.cdiv(v, _NB),)
    body = _tc_proj_body
    in_specs = [
        pl.BlockSpec((_BC, _D), lambda j: (0, 0)),
        pl.BlockSpec((_L, _BC), lambda j: (0, 0)),
        pl.BlockSpec((_BC, 1), lambda j: (0, 0)),
        pl.BlockSpec((1, _D), lambda j: (0, 0)),
        pl.BlockSpec((_NB, _D), lambda j: (j, 0)),
    ]
    args = (sums, contexts_t, lengths2d, w0, W_out_t)
    aliases = {}
    if prev is not None:
        body = lambda p, *rest: _tc_proj_body(*rest)
        in_specs = [pl.BlockSpec(memory_space=pl.ANY)] + in_specs
        args = (prev,) + args
        aliases = {0: 0}
    return pl.pallas_call(
        body,
        grid=grid,
        in_specs=in_specs,
        out_specs=pl.BlockSpec((_NB, _BC), lambda j, c=chunk: (j, c)),
        out_shape=jax.ShapeDtypeStruct((v, _B), jnp.float32),
        scratch_shapes=[pltpu.VMEM((_D, _BC), jnp.bfloat16)],
        input_output_aliases=aliases,
        compiler_params=pltpu.CompilerParams(
            dimension_semantics=("arbitrary",)),
    )(*args)


def kernel(contexts, lengths, W_in, W_out):
    ctx_pad = jnp.concatenate(
        [contexts, jnp.zeros((_B, _LPAD - _L), jnp.int32)], axis=1)
    ctx_flat = ctx_pad.reshape(-1)
    w_flat = W_in.reshape(-1)
    ctx_t = contexts.T
    len2d = lengths.reshape(_B, 1)
    w0 = W_in[0:1]
    w_t = W_out.T

    sums = [_sc_pool(w_flat,
                     lax.slice_in_dim(ctx_flat, c * _BC * _LPAD,
                                      (c + 1) * _BC * _LPAD, axis=0))
            for c in range(_NC)]
    out = None
    for c in range(_NC):
        out = _tc_proj(out, c, sums[c],
                       lax.slice_in_dim(ctx_t, c * _BC, (c + 1) * _BC, axis=1),
                       lax.slice_in_dim(len2d, c * _BC, (c + 1) * _BC, axis=0),
                       w0, w_t)
    return out.T


# speedup vs baseline: 1.2656x; 1.2656x over previous
import functools

import jax
import jax.numpy as jnp
from jax import lax
from jax.experimental import pallas as pl
from jax.experimental.pallas import tpu as pltpu
from jax.experimental.pallas import tpu_sc as plsc

_B = 1024
_L = 50
_LPAD = 56
_D = 128
_NW = 32
_ROWS = _B // _NW
_NBUF = 4
_HALF = _LPAD // 2
_NB = 2048


def _sc_pool_body(w_hbm, ctx_hbm, out_hbm, idx_v, h_v, shared, bufs, obufs,
                  sems, dsems, hsems):
    sid = lax.axis_index("s")
    wid = sid * 2 + lax.axis_index("c")
    base = wid * _ROWS
    pltpu.sync_copy(ctx_hbm.at[pl.ds(base * _LPAD, _ROWS * _LPAD)], idx_v)

    def issue_row(k, b):
        o = k * _LPAD
        vecs = [idx_v[pl.ds(o, 16)], idx_v[pl.ds(o + 16, 16)],
                idx_v[pl.ds(o + 32, 16)], idx_v[pl.ds(o + 40, 16)]]
        for g in range(_LPAD):
            j, l = (g // 16, g % 16) if g < 48 else (3, g - 40)
            r = vecs[j][l]
            if g % 2 == 0:
                pltpu.async_copy(w_hbm.at[pl.ds(r * _D, _D)],
                                 bufs[b].at[pl.ds((g // 2) * _D, _D)],
                                 sems[b])
            else:
                pltpu.async_copy(w_hbm.at[pl.ds(r * _D, _D)],
                                 shared.at[sid, b, pl.ds((g // 2) * _D, _D)],
                                 dsems[b])

    def drain(b):
        pltpu.make_async_copy(w_hbm.at[pl.ds(0, _HALF * _D)], bufs[b],
                              sems[b]).wait()
        pltpu.make_async_copy(w_hbm.at[pl.ds(0, _HALF * _D)],
                              shared.at[sid, b], dsems[b]).wait()
        pltpu.async_copy(shared.at[sid, b], obufs[b], hsems[b]).wait()

    for b in range(_NBUF):
        issue_row(b, b)

    def step(i, carry):
        for b in range(_NBUF):
            k = i * _NBUF + b
            drain(b)

            for c in range(_D // 16):
                acc = bufs[b][pl.ds(c * 16, 16)]
                for g in range(1, _LPAD):
                    src = bufs[b] if g % 2 == 0 else obufs[b]
                    acc = acc + src[pl.ds((g // 2) * _D + c * 16, 16)]
                h_v[k, pl.ds(c * 16, 16)] = acc

            @pl.when(k + _NBUF < _ROWS)
            def _():
                issue_row(k + _NBUF, b)
        return carry

    lax.fori_loop(0, _ROWS // _NBUF, step, 0)
    pltpu.sync_copy(h_v, out_hbm.at[pl.ds(base, _ROWS)])


_sc_pool = functools.partial(
    pl.kernel,
    out_type=jax.ShapeDtypeStruct((_B, _D), jnp.float32),
    mesh=plsc.VectorSubcoreMesh(core_axis_name="c", subcore_axis_name="s"),
    scratch_types=[
        pltpu.VMEM((_ROWS * _LPAD,), jnp.int32),
        pltpu.VMEM((_ROWS, _D), jnp.float32),
        pltpu.VMEM_SHARED((16, _NBUF, _HALF * _D), jnp.float32),
        [pltpu.VMEM((_HALF * _D,), jnp.float32)] * _NBUF,
        [pltpu.VMEM((_HALF * _D,), jnp.float32)] * _NBUF,
        [pltpu.SemaphoreType.DMA] * _NBUF,
        [pltpu.SemaphoreType.DMA] * _NBUF,
        [pltpu.SemaphoreType.DMA] * _NBUF,
    ],
    compiler_params=pltpu.CompilerParams(use_tc_tiling_on_sc=False),
)(_sc_pool_body)


def _tc_proj_body(sums_ref, ctxt_ref, len_ref, w0_ref, wt_ref, out_ref, ht_ref):
    @pl.when(pl.program_id(0) == 0)
    def _():
        n0 = jnp.sum((ctxt_ref[...] == 0).astype(jnp.float32),
                     axis=0)[:, None] + float(_LPAD - _L)
        inv_len = 1.0 / jnp.maximum(len_ref[...], 1).astype(jnp.float32)
        h = (sums_ref[...] - n0 * w0_ref[...]) * inv_len
        ht_ref[...] = jnp.transpose(h).astype(jnp.bfloat16)

    out_ref[...] = jnp.dot(
        wt_ref[...].astype(jnp.bfloat16),
        ht_ref[...],
        preferred_element_type=jnp.float32,
    )


def _tc_proj(sums, contexts_t, lengths2d, w0, W_out_t):
    v = W_out_t.shape[0]
    grid = (pl.cdiv(v, _NB),)
    return pl.pallas_call(
        _tc_proj_body,
        grid=grid,
        in_specs=[
            pl.BlockSpec((_B, _D), lambda j: (0, 0)),
            pl.BlockSpec((_L, _B), lambda j: (0, 0)),
            pl.BlockSpec((_B, 1), lambda j: (0, 0)),
            pl.BlockSpec((1, _D), lambda j: (0, 0)),
            pl.BlockSpec((_NB, _D), lambda j: (j, 0)),
        ],
        out_specs=pl.BlockSpec((_NB, _B), lambda j: (j, 0)),
        out_shape=jax.ShapeDtypeStruct((v, _B), jnp.float32),
        scratch_shapes=[pltpu.VMEM((_D, _B), jnp.bfloat16)],
        compiler_params=pltpu.CompilerParams(
            dimension_semantics=("arbitrary",)),
    )(sums, contexts_t, lengths2d, w0, W_out_t)


def kernel(contexts, lengths, W_in, W_out):
    ctx_pad = jnp.concatenate(
        [contexts, jnp.zeros((_B, _LPAD - _L), jnp.int32)], axis=1)
    sums = _sc_pool(W_in.reshape(-1), ctx_pad.reshape(-1))
    logits_t = _tc_proj(sums, contexts.T, lengths.reshape(_B, 1),
                        W_in[0:1], W_out.T)
    return logits_t.T
